# baseline (speedup 1.0000x reference)
import functools

import jax
import jax.numpy as jnp
from jax import lax
from jax.experimental import pallas as pl
from jax.experimental.pallas import tpu as pltpu
from jax.experimental.pallas import tpu_sc as plsc

B, T, D, WDIM = 16, 32768, 1024, 512
L = 16
NC, NS = 2, 16
NW = NC * NS
CHUNK = 16
NBUF = 2
JV = D // L

T_SC = 11264
TCB = 1024
OFFPAD = 128


def _affine(global_feat, W_affine, b_affine):
    def body(gf_ref, w_ref, b_ref, out_ref):
        out_ref[...] = (
            jnp.dot(gf_ref[...], w_ref[...], preferred_element_type=jnp.float32)
            + b_ref[...]
        )

    return pl.pallas_call(
        body,
        out_shape=jax.ShapeDtypeStruct((B, D), jnp.float32),
    )(global_feat, W_affine, b_affine.reshape(1, D))


def _xlane_sum(x, lane):
    dnums = lax.GatherDimensionNumbers(
        offset_dims=(), collapsed_slice_dims=(0,), start_index_map=(0,))
    for s in (1, 2, 4, 8):
        perm = lane ^ s
        x = x + lax.gather(x, perm[:, None], dnums, slice_sizes=(1,),
                           mode=lax.GatherScatterMode.PROMISE_IN_BOUNDS)
    return x


def _rsqrt(x):
    i = lax.bitcast_convert_type(x, jnp.int32)
    i = jnp.int32(0x5F3759DF) - (i >> 1)
    y = lax.bitcast_convert_type(i, jnp.float32)
    half = x * jnp.float32(0.5)
    for _ in range(3):
        y = y * (jnp.float32(1.5) - half * y * y)
    return y


def _make_sc(rows_total):
    rows_per_w = rows_total // NW
    nchunk = rows_per_w // CHUNK

    @functools.partial(
        pl.kernel,
        mesh=plsc.VectorSubcoreMesh(core_axis_name="c", subcore_axis_name="s"),
        out_type=jax.ShapeDtypeStruct((rows_total, D), jnp.float32),
        scratch_types=[
            pltpu.VMEM((B, D), jnp.float32),
            pltpu.VMEM((B,), jnp.int32),
            pltpu.VMEM((NBUF, CHUNK, D), jnp.float32),
            pltpu.VMEM((NBUF, CHUNK, D), jnp.float32),
            pltpu.SemaphoreType.DMA((NBUF,)),
            pltpu.SemaphoreType.DMA((NBUF,)),
        ],
    )
    def _sc_main(feat_hbm, g_hbm, off_hbm, out_hbm, g_v, off_v, inb, outb,
                 insem, outsem):
        wid = lax.axis_index("s") * NC + lax.axis_index("c")
        base = wid * rows_per_w

        pltpu.sync_copy(g_hbm, g_v)
        pltpu.sync_copy(off_hbm, off_v)
        off_vec = off_v[:]
        lane = lax.iota(jnp.int32, L)
        inv_d = jnp.float32(1.0 / D)

        def in_copy(c, slot):
            return pltpu.make_async_copy(
                feat_hbm.at[pl.ds(base + c * CHUNK, CHUNK)], inb.at[slot],
                insem.at[slot])

        def out_copy(c, slot):
            return pltpu.make_async_copy(
                outb.at[slot], out_hbm.at[pl.ds(base + c * CHUNK, CHUNK)],
                outsem.at[slot])

        in_copy(0, 0).start()

        def chunk_body(c, _):
            slot = lax.rem(c, NBUF)

            @pl.when(c + 1 < nchunk)
            def _prefetch():
                in_copy(c + 1, lax.rem(c + 1, NBUF)).start()

            in_copy(c, slot).wait()

            @pl.when(c >= NBUF)
            def _drain():
                out_copy(c - NBUF, slot).wait()

            def row_body(r, _):
                t = base + c * CHUNK + r
                seg = (-_xlane_sum((off_vec - (t + 1)) >> 31, lane))[0]
                acc = jnp.zeros((L,), jnp.float32)
                acc2 = jnp.zeros((L,), jnp.float32)
                for j in range(JV):
                    v = inb[slot, r, pl.ds(j * L, L)]
                    acc = acc + v
                    acc2 = acc2 + v * v
                mean = _xlane_sum(acc, lane) * inv_d
                ex2 = _xlane_sum(acc2, lane) * inv_d
                var = ex2 - mean * mean
                rinv = _rsqrt(var + jnp.float32(1e-5))
                for j in range(JV):
                    v = inb[slot, r, pl.ds(j * L, L)]
                    gv = g_v[seg, pl.ds(j * L, L)]
                    outb[slot, r, pl.ds(j * L, L)] = (v - mean) * rinv * gv
                return 0

            lax.fori_loop(0, CHUNK, row_body, 0)
            out_copy(c, slot).start()
            return 0

        lax.fori_loop(0, nchunk, chunk_body, 0)
        for k in (nchunk - NBUF, nchunk - 1):
            out_copy(k, k % NBUF).wait()

    return _sc_main


_sc_shard = _make_sc(T_SC)


def _tc_ln(feat, gpad, offp):
    nblk = (T - T_SC) // TCB
    blk0 = T_SC // TCB

    def body(feat_ref, g_ref, off_ref, o_ref):
        i = pl.program_id(0)
        t0 = T_SC + i * TCB
        rows = t0 + lax.broadcasted_iota(jnp.int32, (TCB, 1), 0)
        off_row = off_ref[0:1, :]
        le = (off_row <= rows).astype(jnp.int32)
        seg = jnp.sum(le, axis=1, keepdims=True)
        onehot = (lax.broadcasted_iota(jnp.int32, (TCB, OFFPAD), 1)
                  == seg).astype(jnp.float32)
        grows = jnp.dot(onehot, g_ref[...],
                        preferred_element_type=jnp.float32,
                        precision=lax.Precision.HIGHEST)
        x = feat_ref[...]
        mean = jnp.mean(x, axis=1, keepdims=True)
        ex2 = jnp.mean(x * x, axis=1, keepdims=True)
        var = ex2 - mean * mean
        o_ref[...] = (x - mean) * lax.rsqrt(var + jnp.float32(1e-5)) * grows

    return pl.pallas_call(
        body,
        grid=(nblk,),
        in_specs=[
            pl.BlockSpec((TCB, D), lambda i: (blk0 + i, 0)),
            pl.BlockSpec((OFFPAD, D), lambda i: (0, 0)),
            pl.BlockSpec((8, OFFPAD), lambda i: (0, 0)),
        ],
        out_specs=pl.BlockSpec((TCB, D), lambda i: (blk0 + i, 0)),
        out_shape=jax.ShapeDtypeStruct((T, D), jnp.float32),
    )(feat, gpad, offp)


def kernel(feat, global_feat, offset, W_affine, b_affine):
    g = _affine(global_feat, W_affine, b_affine)
    off = offset.astype(jnp.int32)
    offp = jnp.pad(off[None, :], ((0, 7), (0, OFFPAD - B)),
                   constant_values=jnp.int32(0x7FFFFFFF))
    gpad = jnp.pad(g, ((0, OFFPAD - B), (0, 0)))
    out_tc = _tc_ln(feat, gpad, offp)
    out_sc = _sc_shard(feat, g, off)
    return lax.dynamic_update_slice(out_tc, out_sc, (0, 0))


# speedup vs baseline: 1.0692x; 1.0692x over previous
import functools

import jax
import jax.numpy as jnp
from jax import lax
from jax.experimental import pallas as pl
from jax.experimental.pallas import tpu as pltpu
from jax.experimental.pallas import tpu_sc as plsc

B, T, D, WDIM = 16, 32768, 1024, 512
L = 16
NC, NS = 2, 16
NW = NC * NS
CHUNK = 16
NBUF = 2
JV = D // L

T_SC = 10240
TCB = 1024
OFFPAD = 128


def _affine(global_feat, W_affine, b_affine):
    def body(gf_ref, w_ref, b_ref, out_ref):
        out_ref[...] = (
            jnp.dot(gf_ref[...], w_ref[...], preferred_element_type=jnp.float32)
            + b_ref[...]
        )

    return pl.pallas_call(
        body,
        out_shape=jax.ShapeDtypeStruct((B, D), jnp.float32),
    )(global_feat, W_affine, b_affine.reshape(1, D))


def _xlane_sum(x, lane):
    dnums = lax.GatherDimensionNumbers(
        offset_dims=(), collapsed_slice_dims=(0,), start_index_map=(0,))
    for s in (1, 2, 4, 8):
        perm = lane ^ s
        x = x + lax.gather(x, perm[:, None], dnums, slice_sizes=(1,),
                           mode=lax.GatherScatterMode.PROMISE_IN_BOUNDS)
    return x


def _rsqrt(x):
    i = lax.bitcast_convert_type(x, jnp.int32)
    i = jnp.int32(0x5F3759DF) - (i >> 1)
    y = lax.bitcast_convert_type(i, jnp.float32)
    half = x * jnp.float32(0.5)
    for _ in range(3):
        y = y * (jnp.float32(1.5) - half * y * y)
    return y


def _make_sc(rows_total):
    rows_per_w = rows_total // NW
    nchunk = rows_per_w // CHUNK

    @functools.partial(
        pl.kernel,
        mesh=plsc.VectorSubcoreMesh(core_axis_name="c", subcore_axis_name="s"),
        out_type=jax.ShapeDtypeStruct((rows_total, D), jnp.float32),
        scratch_types=[
            pltpu.VMEM((B, D), jnp.float32),
            pltpu.VMEM((B,), jnp.int32),
            pltpu.VMEM((NBUF, CHUNK, D), jnp.float32),
            pltpu.VMEM((NBUF, CHUNK, D), jnp.float32),
            pltpu.SemaphoreType.DMA((NBUF,)),
            pltpu.SemaphoreType.DMA((NBUF,)),
        ],
    )
    def _sc_main(feat_hbm, g_hbm, off_hbm, out_hbm, g_v, off_v, inb, outb,
                 insem, outsem):
        wid = lax.axis_index("s") * NC + lax.axis_index("c")
        base = wid * rows_per_w

        pltpu.sync_copy(g_hbm, g_v)
        pltpu.sync_copy(off_hbm, off_v)
        off_vec = off_v[:]
        lane = lax.iota(jnp.int32, L)
        inv_d = jnp.float32(1.0 / D)

        def in_copy(c, slot):
            return pltpu.make_async_copy(
                feat_hbm.at[pl.ds(base + c * CHUNK, CHUNK)], inb.at[slot],
                insem.at[slot])

        def out_copy(c, slot):
            return pltpu.make_async_copy(
                outb.at[slot], out_hbm.at[pl.ds(base + c * CHUNK, CHUNK)],
                outsem.at[slot])

        in_copy(0, 0).start()

        def chunk_body(c, _):
            slot = lax.rem(c, NBUF)

            @pl.when(c + 1 < nchunk)
            def _prefetch():
                in_copy(c + 1, lax.rem(c + 1, NBUF)).start()

            in_copy(c, slot).wait()

            @pl.when(c >= NBUF)
            def _drain():
                out_copy(c - NBUF, slot).wait()

            def row_body(r, _):
                t = base + c * CHUNK + r
                seg = (-_xlane_sum((off_vec - (t + 1)) >> 31, lane))[0]
                acc = jnp.zeros((L,), jnp.float32)
                acc2 = jnp.zeros((L,), jnp.float32)
                for j in range(JV):
                    v = inb[slot, r, pl.ds(j * L, L)]
                    acc = acc + v
                    acc2 = acc2 + v * v
                mean = _xlane_sum(acc, lane) * inv_d
                ex2 = _xlane_sum(acc2, lane) * inv_d
                var = ex2 - mean * mean
                rinv = _rsqrt(var + jnp.float32(1e-5))
                for j in range(JV):
                    v = inb[slot, r, pl.ds(j * L, L)]
                    gv = g_v[seg, pl.ds(j * L, L)]
                    outb[slot, r, pl.ds(j * L, L)] = (v - mean) * rinv * gv
                return 0

            lax.fori_loop(0, CHUNK, row_body, 0)
            out_copy(c, slot).start()
            return 0

        lax.fori_loop(0, nchunk, chunk_body, 0)
        for k in (nchunk - NBUF, nchunk - 1):
            out_copy(k, k % NBUF).wait()

    return _sc_main


_sc_shard = _make_sc(T_SC)


def _tc_ln(feat, gpad, offp):
    nblk = (T - T_SC) // TCB
    blk0 = T_SC // TCB

    def body(feat_ref, g_ref, off_ref, o_ref):
        i = pl.program_id(0)
        t0 = T_SC + i * TCB
        rows = t0 + lax.broadcasted_iota(jnp.int32, (TCB, 1), 0)
        off_row = off_ref[0:1, :]
        le = (off_row <= rows).astype(jnp.int32)
        seg = jnp.sum(le, axis=1, keepdims=True)
        onehot = (lax.broadcasted_iota(jnp.int32, (TCB, OFFPAD), 1)
                  == seg).astype(jnp.float32)
        grows = jnp.dot(onehot, g_ref[...],
                        preferred_element_type=jnp.float32,
                        precision=lax.Precision.HIGHEST)
        x = feat_ref[...]
        mean = jnp.mean(x, axis=1, keepdims=True)
        ex2 = jnp.mean(x * x, axis=1, keepdims=True)
        var = ex2 - mean * mean
        o_ref[...] = (x - mean) * lax.rsqrt(var + jnp.float32(1e-5)) * grows

    return pl.pallas_call(
        body,
        grid=(nblk,),
        in_specs=[
            pl.BlockSpec((TCB, D), lambda i: (blk0 + i, 0)),
            pl.BlockSpec((OFFPAD, D), lambda i: (0, 0)),
            pl.BlockSpec((8, OFFPAD), lambda i: (0, 0)),
        ],
        out_specs=pl.BlockSpec((TCB, D), lambda i: (blk0 + i, 0)),
        out_shape=jax.ShapeDtypeStruct((T, D), jnp.float32),
    )(feat, gpad, offp)


def kernel(feat, global_feat, offset, W_affine, b_affine):
    g = _affine(global_feat, W_affine, b_affine)
    off = offset.astype(jnp.int32)
    offp = jnp.pad(off[None, :], ((0, 7), (0, OFFPAD - B)),
                   constant_values=jnp.int32(0x7FFFFFFF))
    gpad = jnp.pad(g, ((0, OFFPAD - B), (0, 0)))
    out_tc = _tc_ln(feat, gpad, offp)
    out_sc = _sc_shard(feat, g, off)
    return lax.dynamic_update_slice(out_tc, out_sc, (0, 0))
